# baseline (speedup 1.0000x reference)
import functools

import jax
import jax.numpy as jnp
from jax import lax
from jax.experimental import pallas as pl
from jax.experimental.pallas import tpu as pltpu
from jax.experimental.pallas import tpu_sc as plsc

N = 10000
NP = 10240
E = 320000
CH = 128
NC, NS = 2, 16
NW = NC * NS
CPT = ((-(-E // (CH * NW)) + 7) // 8) * 8
EP = CPT * CH * NW
CPT2 = CPT // 2

FAST_C = 0
CPT_F = 160
WIN = 32
RPS = NP // NS
BLK = 1024
GRID = NP // BLK


def _sc_mesh():
    return plsc.VectorSubcoreMesh(
        core_axis_name="c", subcore_axis_name="s",
        num_cores=NC, num_subcores=NS)


@functools.partial(
    pl.kernel,
    out_type=jax.ShapeDtypeStruct((NC, NP), jnp.float32),
    mesh=_sc_mesh(),
    scratch_types=[
        pltpu.VMEM((CPT, CH), jnp.int32),
        pltpu.VMEM((CH,), jnp.float32),
        pltpu.VMEM_SHARED((NP,), jnp.float32),
    ],
)
def _deg_kernel(dst_hbm, zeros_hbm, out_hbm, idx_v, ones_v, acc):
    c = lax.axis_index("c")
    s = lax.axis_index("s")
    w = c * NS + s
    pltpu.sync_copy(zeros_hbm.at[pl.ds(s * RPS, RPS)],
                    acc.at[pl.ds(s * RPS, RPS)])
    for i in range(CH // 16):
        ones_v[pl.ds(i * 16, 16)] = jnp.full((16,), 1.0, jnp.float32)
    pltpu.sync_copy(dst_hbm.at[pl.ds(w * CPT, CPT)], idx_v)
    plsc.subcore_barrier()

    def step(j, carry):
        pltpu.sync_copy(ones_v, acc.at[idx_v.at[j]], add=True)
        return carry

    lax.fori_loop(0, CPT, step, 0)
    plsc.subcore_barrier()
    pltpu.sync_copy(acc.at[pl.ds(s * RPS, RPS)],
                    out_hbm.at[c, pl.ds(s * RPS, RPS)])


def _make_agg(F):
    @functools.partial(
        pl.kernel,
        out_type=jax.ShapeDtypeStruct((NP, F), jnp.float32),
        mesh=_sc_mesh(),
        scratch_types=[
            pltpu.VMEM((WIN, CH), jnp.int32),
            pltpu.VMEM((WIN, CH), jnp.int32),
            pltpu.VMEM((2, CH, F), jnp.float32),
            pltpu.VMEM_SHARED((NP, F), jnp.float32),
            pltpu.SemaphoreType.DMA,
            pltpu.SemaphoreType.DMA,
        ],
    )
    def _agg(u_hbm, src_hbm, dst_hbm, zeros_hbm, out_hbm,
             si_v, di_v, rows_v, acc, gsem, ssem):
        c = lax.axis_index("c")
        s = lax.axis_index("s")
        is_fast = c == FAST_C
        tile_base = s * CPT_F
        n_win = jnp.where(is_fast, CPT_F // WIN, 0)

        @pl.when(is_fast)
        def _():
            pltpu.sync_copy(zeros_hbm.at[pl.ds(s * RPS, RPS)],
                            acc.at[pl.ds(s * RPS, RPS)])
        plsc.subcore_barrier()

        def window(h, carry0):
            base = pl.multiple_of(tile_base + h * WIN, 8)
            pltpu.sync_copy(src_hbm.at[pl.ds(base, WIN)], si_v)
            pltpu.sync_copy(dst_hbm.at[pl.ds(base, WIN)], di_v)

            pltpu.async_copy(u_hbm.at[si_v.at[0]], rows_v.at[0], gsem)

            def step(j, carry):
                p = lax.rem(j, 2)
                q = 1 - p
                pltpu.make_async_copy(
                    u_hbm.at[si_v.at[j]], rows_v.at[p], gsem).wait()

                @pl.when(j >= 1)
                def _():
                    pltpu.make_async_copy(
                        rows_v.at[q], acc.at[di_v.at[j]], ssem).wait()

                @pl.when(j + 1 < WIN)
                def _():
                    pltpu.async_copy(
                        u_hbm.at[si_v.at[j + 1]], rows_v.at[q], gsem)

                pltpu.async_copy(
                    rows_v.at[p], acc.at[di_v.at[j]], ssem, add=True)
                return carry

            lax.fori_loop(0, WIN, step, 0)
            pltpu.make_async_copy(rows_v.at[0], acc.at[di_v.at[0]], ssem).wait()
            return carry0

        lax.fori_loop(0, n_win, window, 0)
        plsc.subcore_barrier()

        @pl.when(is_fast)
        def _():
            pltpu.sync_copy(acc.at[pl.ds(s * RPS, RPS)],
                            out_hbm.at[pl.ds(s * RPS, RPS)])

    return _agg


def _tc1_body(degp_ref, x_ref, w1_ref, u1_ref, d_ref):
    deg = degp_ref[:, 0:1] + degp_ref[:, 1:2] + 1.0
    d = lax.rsqrt(deg)
    u1_ref[...] = jnp.dot(x_ref[...], w1_ref[...],
                          preferred_element_type=jnp.float32) * d
    d_ref[...] = d


def _tc2_body(q_ref, u1_ref, d_ref, b1_ref, v_ref):
    t = q_ref[...] + u1_ref[...]
    h = jnp.maximum(t * d_ref[...] + b1_ref[...], 0.0)
    v_ref[...] = h * d_ref[...]


def _tc3_body(r_ref, v_ref, d_ref, b2_ref, w2_ref, out_ref):
    t = (r_ref[...] + v_ref[...]) * d_ref[...]
    out_ref[...] = jnp.dot(t, w2_ref[...],
                           preferred_element_type=jnp.float32) + b2_ref[...]


def _tc1(degp_t, x_p, W1):
    H = W1.shape[1]
    return pl.pallas_call(
        _tc1_body,
        grid=(GRID,),
        in_specs=[
            pl.BlockSpec((BLK, NC), lambda i: (i, 0)),
            pl.BlockSpec((BLK, x_p.shape[1]), lambda i: (i, 0)),
            pl.BlockSpec(W1.shape, lambda i: (0, 0)),
        ],
        out_specs=[
            pl.BlockSpec((BLK, H), lambda i: (i, 0)),
            pl.BlockSpec((BLK, 1), lambda i: (i, 0)),
        ],
        out_shape=[
            jax.ShapeDtypeStruct((NP, H), jnp.float32),
            jax.ShapeDtypeStruct((NP, 1), jnp.float32),
        ],
    )(degp_t, x_p, W1)


def _tc2(q, u1, d, b1r):
    H = u1.shape[1]
    return pl.pallas_call(
        _tc2_body,
        grid=(GRID,),
        in_specs=[
            pl.BlockSpec((BLK, H), lambda i: (i, 0)),
            pl.BlockSpec((BLK, H), lambda i: (i, 0)),
            pl.BlockSpec((BLK, 1), lambda i: (i, 0)),
            pl.BlockSpec((1, H), lambda i: (0, 0)),
        ],
        out_specs=pl.BlockSpec((BLK, H), lambda i: (i, 0)),
        out_shape=jax.ShapeDtypeStruct((NP, H), jnp.float32),
    )(q, u1, d, b1r)


def _tc3(r, v, d, b2r, W2):
    H = v.shape[1]
    O = W2.shape[1]
    return pl.pallas_call(
        _tc3_body,
        grid=(GRID,),
        in_specs=[
            pl.BlockSpec((BLK, H), lambda i: (i, 0)),
            pl.BlockSpec((BLK, H), lambda i: (i, 0)),
            pl.BlockSpec((BLK, 1), lambda i: (i, 0)),
            pl.BlockSpec((1, O), lambda i: (0, 0)),
            pl.BlockSpec(W2.shape, lambda i: (0, 0)),
        ],
        out_specs=pl.BlockSpec((BLK, O), lambda i: (i, 0)),
        out_shape=jax.ShapeDtypeStruct((NP, O), jnp.float32),
    )(r, v, d, b2r, W2)


def kernel(x, edge_index, W1, b1, W2, b2):
    H = W1.shape[1]
    O = W2.shape[1]
    src = edge_index[0]
    dst = edge_index[1]
    pad_e = EP - E
    src_p = jnp.concatenate(
        [src, jnp.zeros((pad_e,), jnp.int32)]).reshape(NW * CPT, CH)
    pad_dst = N + (jnp.arange(pad_e, dtype=jnp.int32) % (NP - N))
    dst_p = jnp.concatenate([dst, pad_dst]).reshape(NW * CPT, CH)
    x_p = jnp.pad(x, ((0, NP - N), (0, 0)))

    zeros1 = jnp.zeros((NP,), jnp.float32)
    zerosH = jnp.zeros((NP, H), jnp.float32)
    agg = _make_agg(H)

    degp = _deg_kernel(dst_p, zeros1)
    u1, d = _tc1(degp.T, x_p, W1)
    q = agg(u1, src_p, dst_p, zerosH)
    v = _tc2(q, u1, d, b1.reshape(1, H))
    r = agg(v, src_p, dst_p, zerosH)
    out = _tc3(r, v, d, b2.reshape(1, O), W2)
    return out[:N]


# speedup vs baseline: 2.4348x; 2.4348x over previous
import functools

import jax
import jax.numpy as jnp
from jax import lax
from jax.experimental import pallas as pl
from jax.experimental.pallas import tpu as pltpu
from jax.experimental.pallas import tpu_sc as plsc

N = 10000
NP = 10240
E = 320000
CH = 128
NC, NS = 2, 16
NW = NC * NS
CPT = ((-(-E // (CH * NW)) + 7) // 8) * 8
EP = CPT * CH * NW
CPT2 = CPT // 2

FAST_C = 0
CPT_F = 160
WIN = 32
RPS = NP // NS
BLK = 1024
GRID = NP // BLK


def _sc_mesh():
    return plsc.VectorSubcoreMesh(
        core_axis_name="c", subcore_axis_name="s",
        num_cores=NC, num_subcores=NS)


@functools.partial(
    pl.kernel,
    out_type=jax.ShapeDtypeStruct((NC, NP), jnp.float32),
    mesh=_sc_mesh(),
    scratch_types=[
        pltpu.VMEM((CPT, CH), jnp.int32),
        pltpu.VMEM((CH,), jnp.float32),
        pltpu.VMEM_SHARED((NP,), jnp.float32),
    ],
)
def _deg_kernel(dst_hbm, zeros_hbm, out_hbm, idx_v, ones_v, acc):
    c = lax.axis_index("c")
    s = lax.axis_index("s")
    w = c * NS + s
    pltpu.sync_copy(zeros_hbm.at[pl.ds(s * RPS, RPS)],
                    acc.at[pl.ds(s * RPS, RPS)])
    for i in range(CH // 16):
        ones_v[pl.ds(i * 16, 16)] = jnp.full((16,), 1.0, jnp.float32)
    pltpu.sync_copy(dst_hbm.at[pl.ds(w * CPT, CPT)], idx_v)
    plsc.subcore_barrier()

    def step(j, carry):
        pltpu.sync_copy(ones_v, acc.at[idx_v.at[j]], add=True)
        return carry

    lax.fori_loop(0, CPT, step, 0)
    plsc.subcore_barrier()
    pltpu.sync_copy(acc.at[pl.ds(s * RPS, RPS)],
                    out_hbm.at[c, pl.ds(s * RPS, RPS)])


def _make_agg(F):
    @functools.partial(
        pl.kernel,
        out_type=jax.ShapeDtypeStruct((NP, F), jnp.float32),
        mesh=_sc_mesh(),
        scratch_types=[
            pltpu.VMEM((WIN, CH), jnp.int32),
            pltpu.VMEM((WIN, CH), jnp.int32),
            pltpu.VMEM((2, CH, F), jnp.float32),
            pltpu.VMEM_SHARED((NP, F), jnp.float32),
            pltpu.SemaphoreType.DMA,
            pltpu.SemaphoreType.DMA,
        ],
    )
    def _agg(u_hbm, src_hbm, dst_hbm, zeros_hbm, out_hbm,
             si_v, di_v, rows_v, acc, gsem, ssem):
        c = lax.axis_index("c")
        s = lax.axis_index("s")
        is_fast = c == FAST_C
        tile_base = s * CPT_F
        n_win = jnp.where(is_fast, CPT_F // WIN, 0)

        @pl.when(is_fast)
        def _():
            pltpu.sync_copy(zeros_hbm.at[pl.ds(s * RPS, RPS)],
                            acc.at[pl.ds(s * RPS, RPS)])
        plsc.subcore_barrier()

        def window(h, carry0):
            base = pl.multiple_of(tile_base + h * WIN, 8)
            pltpu.sync_copy(src_hbm.at[pl.ds(base, WIN)], si_v)
            pltpu.sync_copy(dst_hbm.at[pl.ds(base, WIN)], di_v)

            pltpu.async_copy(u_hbm.at[si_v.at[0]], rows_v.at[0], gsem)

            def step(j, carry):
                p = lax.rem(j, 2)
                q = 1 - p
                pltpu.make_async_copy(
                    u_hbm.at[si_v.at[j]], rows_v.at[p], gsem).wait()

                @pl.when(j >= 1)
                def _():
                    pltpu.make_async_copy(
                        rows_v.at[q], acc.at[di_v.at[j]], ssem).wait()

                @pl.when(j + 1 < WIN)
                def _():
                    pltpu.async_copy(
                        u_hbm.at[si_v.at[j + 1]], rows_v.at[q], gsem)

                pltpu.async_copy(
                    rows_v.at[p], acc.at[di_v.at[j]], ssem, add=True)
                return carry

            lax.fori_loop(0, WIN, step, 0)
            pltpu.make_async_copy(rows_v.at[0], acc.at[di_v.at[0]], ssem).wait()
            return carry0

        lax.fori_loop(0, n_win, window, 0)
        plsc.subcore_barrier()

        @pl.when(is_fast)
        def _():
            pltpu.sync_copy(acc.at[pl.ds(s * RPS, RPS)],
                            out_hbm.at[pl.ds(s * RPS, RPS)])

    return _agg


def _tc1_body(degp_ref, x_ref, w1_ref, u1_ref, d_ref):
    deg = degp_ref[:, 0:1] + degp_ref[:, 1:2] + 1.0
    d = lax.rsqrt(deg)
    u1_ref[...] = jnp.dot(x_ref[...], w1_ref[...],
                          preferred_element_type=jnp.float32) * d
    d_ref[...] = d


def _tc2_body(q_ref, u1_ref, d_ref, b1_ref, v_ref):
    t = q_ref[...] + u1_ref[...]
    h = jnp.maximum(t * d_ref[...] + b1_ref[...], 0.0)
    v_ref[...] = h * d_ref[...]


def _tc3_body(r_ref, v_ref, d_ref, b2_ref, w2_ref, out_ref):
    t = (r_ref[...] + v_ref[...]) * d_ref[...]
    out_ref[...] = jnp.dot(t, w2_ref[...],
                           preferred_element_type=jnp.float32) + b2_ref[...]


def _tc1(degp_t, x_p, W1):
    H = W1.shape[1]
    return pl.pallas_call(
        _tc1_body,
        grid=(GRID,),
        in_specs=[
            pl.BlockSpec((BLK, NC), lambda i: (i, 0)),
            pl.BlockSpec((BLK, x_p.shape[1]), lambda i: (i, 0)),
            pl.BlockSpec(W1.shape, lambda i: (0, 0)),
        ],
        out_specs=[
            pl.BlockSpec((BLK, H), lambda i: (i, 0)),
            pl.BlockSpec((BLK, 1), lambda i: (i, 0)),
        ],
        out_shape=[
            jax.ShapeDtypeStruct((NP, H), jnp.float32),
            jax.ShapeDtypeStruct((NP, 1), jnp.float32),
        ],
    )(degp_t, x_p, W1)


def _tc2(q, u1, d, b1r):
    H = u1.shape[1]
    return pl.pallas_call(
        _tc2_body,
        grid=(GRID,),
        in_specs=[
            pl.BlockSpec((BLK, H), lambda i: (i, 0)),
            pl.BlockSpec((BLK, H), lambda i: (i, 0)),
            pl.BlockSpec((BLK, 1), lambda i: (i, 0)),
            pl.BlockSpec((1, H), lambda i: (0, 0)),
        ],
        out_specs=pl.BlockSpec((BLK, H), lambda i: (i, 0)),
        out_shape=jax.ShapeDtypeStruct((NP, H), jnp.float32),
    )(q, u1, d, b1r)


def _tc3(r, v, d, b2r, W2):
    H = v.shape[1]
    O = W2.shape[1]
    return pl.pallas_call(
        _tc3_body,
        grid=(GRID,),
        in_specs=[
            pl.BlockSpec((BLK, H), lambda i: (i, 0)),
            pl.BlockSpec((BLK, H), lambda i: (i, 0)),
            pl.BlockSpec((BLK, 1), lambda i: (i, 0)),
            pl.BlockSpec((1, O), lambda i: (0, 0)),
            pl.BlockSpec(W2.shape, lambda i: (0, 0)),
        ],
        out_specs=pl.BlockSpec((BLK, O), lambda i: (i, 0)),
        out_shape=jax.ShapeDtypeStruct((NP, O), jnp.float32),
    )(r, v, d, b2r, W2)


def kernel(x, edge_index, W1, b1, W2, b2):
    H = W1.shape[1]
    O = W2.shape[1]
    src = edge_index[0]
    dst = edge_index[1]
    pad_e = EP - E
    pad_src = jnp.arange(pad_e, dtype=jnp.int32) % N
    src_p = jnp.concatenate([src, pad_src]).reshape(NW * CPT, CH)
    pad_dst = N + (jnp.arange(pad_e, dtype=jnp.int32) % (NP - N))
    dst_p = jnp.concatenate([dst, pad_dst]).reshape(NW * CPT, CH)
    x_p = jnp.pad(x, ((0, NP - N), (0, 0)))

    zeros1 = jnp.zeros((NP,), jnp.float32)
    zerosH = jnp.zeros((NP, H), jnp.float32)
    agg = _make_agg(H)

    degp = _deg_kernel(dst_p, zeros1)
    u1, d = _tc1(degp.T, x_p, W1)
    q = agg(u1, src_p, dst_p, zerosH)
    v = _tc2(q, u1, d, b1.reshape(1, H))
    r = agg(v, src_p, dst_p, zerosH)
    out = _tc3(r, v, d, b2.reshape(1, O), W2)
    return out[:N]


# speedup vs baseline: 3.9190x; 1.6096x over previous
import functools

import jax
import jax.numpy as jnp
from jax import lax
from jax.experimental import pallas as pl
from jax.experimental.pallas import tpu as pltpu
from jax.experimental.pallas import tpu_sc as plsc

N = 10000
NP = 10240
E = 320000
CH = 128
NC, NS = 2, 16
NW = NC * NS
CPT = ((-(-E // (CH * NW)) + 7) // 8) * 8
EP = CPT * CH * NW
CPT2 = CPT // 2

WIN = 40
RPS = NP // NS
BLK = 1024
GRID = NP // BLK


def _sc_mesh():
    return plsc.VectorSubcoreMesh(
        core_axis_name="c", subcore_axis_name="s",
        num_cores=NC, num_subcores=NS)


@functools.partial(
    pl.kernel,
    out_type=jax.ShapeDtypeStruct((NC, NP), jnp.float32),
    mesh=_sc_mesh(),
    scratch_types=[
        pltpu.VMEM((CPT, CH), jnp.int32),
        pltpu.VMEM((CH,), jnp.float32),
        pltpu.VMEM_SHARED((NP,), jnp.float32),
    ],
)
def _deg_kernel(dst_hbm, zeros_hbm, out_hbm, idx_v, ones_v, acc):
    c = lax.axis_index("c")
    s = lax.axis_index("s")
    w = c * NS + s
    pltpu.sync_copy(zeros_hbm.at[pl.ds(s * RPS, RPS)],
                    acc.at[pl.ds(s * RPS, RPS)])
    for i in range(CH // 16):
        ones_v[pl.ds(i * 16, 16)] = jnp.full((16,), 1.0, jnp.float32)
    pltpu.sync_copy(dst_hbm.at[pl.ds(w * CPT, CPT)], idx_v)
    plsc.subcore_barrier()

    def step(j, carry):
        pltpu.sync_copy(ones_v, acc.at[idx_v.at[j]], add=True)
        return carry

    lax.fori_loop(0, CPT, step, 0)
    plsc.subcore_barrier()
    pltpu.sync_copy(acc.at[pl.ds(s * RPS, RPS)],
                    out_hbm.at[c, pl.ds(s * RPS, RPS)])


def _make_agg(F):
    @functools.partial(
        pl.kernel,
        out_type=jax.ShapeDtypeStruct((NC, NP, F), jnp.float32),
        mesh=_sc_mesh(),
        scratch_types=[
            pltpu.VMEM((WIN, CH), jnp.int32),
            pltpu.VMEM((WIN, CH), jnp.int32),
            pltpu.VMEM((2, CH, F), jnp.float32),
            pltpu.VMEM_SHARED((NP, F), jnp.float32),
            pltpu.SemaphoreType.DMA,
            pltpu.SemaphoreType.DMA,
        ],
    )
    def _agg(u_hbm, src_hbm, dst_hbm, zeros_hbm, out_hbm,
             si_v, di_v, rows_v, acc, gsem, ssem):
        c = lax.axis_index("c")
        s = lax.axis_index("s")
        tile_base = (c * NS + s) * CPT
        pltpu.sync_copy(zeros_hbm.at[pl.ds(s * RPS, RPS)],
                        acc.at[pl.ds(s * RPS, RPS)])
        plsc.subcore_barrier()

        def window(h, carry0):
            base = pl.multiple_of(tile_base + h * WIN, 8)
            pltpu.sync_copy(src_hbm.at[pl.ds(base, WIN)], si_v)
            pltpu.sync_copy(dst_hbm.at[pl.ds(base, WIN)], di_v)

            pltpu.async_copy(u_hbm.at[si_v.at[0]], rows_v.at[0], gsem)

            def step(j, carry):
                p = lax.rem(j, 2)
                q = 1 - p
                pltpu.make_async_copy(
                    u_hbm.at[si_v.at[j]], rows_v.at[p], gsem).wait()

                @pl.when(j >= 1)
                def _():
                    pltpu.make_async_copy(
                        rows_v.at[q], acc.at[di_v.at[j]], ssem).wait()

                @pl.when(j + 1 < WIN)
                def _():
                    pltpu.async_copy(
                        u_hbm.at[si_v.at[j + 1]], rows_v.at[q], gsem)

                pltpu.async_copy(
                    rows_v.at[p], acc.at[di_v.at[j]], ssem, add=True)
                return carry

            lax.fori_loop(0, WIN, step, 0)
            pltpu.make_async_copy(rows_v.at[0], acc.at[di_v.at[0]], ssem).wait()
            return carry0

        lax.fori_loop(0, CPT // WIN, window, 0)
        plsc.subcore_barrier()
        pltpu.sync_copy(acc.at[pl.ds(s * RPS, RPS)],
                        out_hbm.at[c, pl.ds(s * RPS, RPS)])

    return _agg


def _tc1_body(degp_ref, x_ref, w1_ref, u1_ref, d_ref):
    deg = degp_ref[:, 0:1] + degp_ref[:, 1:2] + 1.0
    d = lax.rsqrt(deg)
    u1_ref[...] = jnp.dot(x_ref[...], w1_ref[...],
                          preferred_element_type=jnp.float32) * d
    d_ref[...] = d


def _tc2_body(q0_ref, q1_ref, u1_ref, d_ref, b1_ref, v_ref):
    t = q0_ref[0] + q1_ref[0] + u1_ref[...]
    h = jnp.maximum(t * d_ref[...] + b1_ref[...], 0.0)
    v_ref[...] = h * d_ref[...]


def _tc3_body(r0_ref, r1_ref, v_ref, d_ref, b2_ref, w2_ref, out_ref):
    t = (r0_ref[0] + r1_ref[0] + v_ref[...]) * d_ref[...]
    out_ref[...] = jnp.dot(t, w2_ref[...],
                           preferred_element_type=jnp.float32) + b2_ref[...]


def _tc1(degp_t, x_p, W1):
    H = W1.shape[1]
    return pl.pallas_call(
        _tc1_body,
        grid=(GRID,),
        in_specs=[
            pl.BlockSpec((BLK, NC), lambda i: (i, 0)),
            pl.BlockSpec((BLK, x_p.shape[1]), lambda i: (i, 0)),
            pl.BlockSpec(W1.shape, lambda i: (0, 0)),
        ],
        out_specs=[
            pl.BlockSpec((BLK, H), lambda i: (i, 0)),
            pl.BlockSpec((BLK, 1), lambda i: (i, 0)),
        ],
        out_shape=[
            jax.ShapeDtypeStruct((NP, H), jnp.float32),
            jax.ShapeDtypeStruct((NP, 1), jnp.float32),
        ],
    )(degp_t, x_p, W1)


def _tc2(q, u1, d, b1r):
    H = u1.shape[1]
    return pl.pallas_call(
        _tc2_body,
        grid=(GRID,),
        in_specs=[
            pl.BlockSpec((1, BLK, H), lambda i: (0, i, 0)),
            pl.BlockSpec((1, BLK, H), lambda i: (1, i, 0)),
            pl.BlockSpec((BLK, H), lambda i: (i, 0)),
            pl.BlockSpec((BLK, 1), lambda i: (i, 0)),
            pl.BlockSpec((1, H), lambda i: (0, 0)),
        ],
        out_specs=pl.BlockSpec((BLK, H), lambda i: (i, 0)),
        out_shape=jax.ShapeDtypeStruct((NP, H), jnp.float32),
    )(q, q, u1, d, b1r)


def _tc3(r, v, d, b2r, W2):
    H = v.shape[1]
    O = W2.shape[1]
    return pl.pallas_call(
        _tc3_body,
        grid=(GRID,),
        in_specs=[
            pl.BlockSpec((1, BLK, H), lambda i: (0, i, 0)),
            pl.BlockSpec((1, BLK, H), lambda i: (1, i, 0)),
            pl.BlockSpec((BLK, H), lambda i: (i, 0)),
            pl.BlockSpec((BLK, 1), lambda i: (i, 0)),
            pl.BlockSpec((1, O), lambda i: (0, 0)),
            pl.BlockSpec(W2.shape, lambda i: (0, 0)),
        ],
        out_specs=pl.BlockSpec((BLK, O), lambda i: (i, 0)),
        out_shape=jax.ShapeDtypeStruct((NP, O), jnp.float32),
    )(r, r, v, d, b2r, W2)


def kernel(x, edge_index, W1, b1, W2, b2):
    H = W1.shape[1]
    O = W2.shape[1]
    src = edge_index[0]
    dst = edge_index[1]
    pad_e = EP - E
    pad_src = jnp.arange(pad_e, dtype=jnp.int32) % N
    src_p = jnp.concatenate([src, pad_src]).reshape(NW * CPT, CH)
    pad_dst = N + (jnp.arange(pad_e, dtype=jnp.int32) % (NP - N))
    dst_p = jnp.concatenate([dst, pad_dst]).reshape(NW * CPT, CH)
    x_p = jnp.pad(x, ((0, NP - N), (0, 0)))

    zeros1 = jnp.zeros((NP,), jnp.float32)
    zerosH = jnp.zeros((NP, H), jnp.float32)
    agg = _make_agg(H)

    degp = _deg_kernel(dst_p, zeros1)
    u1, d = _tc1(degp.T, x_p, W1)
    q = agg(u1, src_p, dst_p, zerosH)
    v = _tc2(q, u1, d, b1.reshape(1, H))
    r = agg(v, src_p, dst_p, zerosH)
    out = _tc3(r, v, d, b2.reshape(1, O), W2)
    return out[:N]
